# baseline (speedup 1.0000x reference)
import jax
import jax.numpy as jnp
from jax.experimental import pallas as pl


def _silu(v):
    return v * jax.nn.sigmoid(v)


def _identity_pallas(x):
    def body(x_ref, o_ref):
        o_ref[...] = x_ref[...]
    return pl.pallas_call(
        body, out_shape=jax.ShapeDtypeStruct(x.shape, x.dtype))(x)


def kernel(x, edge_index, edge_attr, params):
    src, dst = edge_index[0], edge_index[1]
    n = x.shape[0]
    deg = jnp.zeros((n,), jnp.float32).at[dst].add(1.0) + 1.0
    dinv = deg ** -0.5
    norm = (dinv[src] * dinv[dst])[:, None]
    h = x
    for i, (W, b) in enumerate(params["gcn"]):
        hW = h @ W
        out = jax.ops.segment_sum(hW[src] * norm, dst, num_segments=n)
        out = out + hW * (dinv ** 2)[:, None] + b
        h = jax.nn.relu(out) if i < len(params["gcn"]) - 1 else out
    coord = x
    for p in params["egcl"]:
        diff = coord[src] - coord[dst]
        radial = jnp.sum(diff * diff, axis=1, keepdims=True)
        ef = jnp.concatenate([h[src], h[dst], radial, edge_attr], axis=1)
        m = _silu(_silu(ef @ p["eW1"] + p["eb1"]) @ p["eW2"] + p["eb2"])
        phi_x = _silu(m @ p["cW1"] + p["cb1"]) @ p["cW2"]
        trans = diff * phi_x
        cnt = jnp.zeros((n, 1), jnp.float32).at[src].add(1.0)
        coord = coord + jax.ops.segment_sum(trans, src, num_segments=n) / jnp.clip(cnt, 1.0, None)
        agg = jax.ops.segment_sum(m, src, num_segments=n)
        out = _silu(jnp.concatenate([h, agg], axis=1) @ p["nW1"] + p["nb1"]) @ p["nW2"] + p["nb2"]
        h = h + out
    return _identity_pallas(h)


# speedup vs baseline: 1.8448x; 1.8448x over previous
import jax
import jax.numpy as jnp
from jax import lax
from jax.experimental import pallas as pl
from jax.experimental.pallas import tpu as pltpu
from jax.experimental.pallas import tpu_sc as plsc

F32 = jnp.float32
I32 = jnp.int32

_N = 10000
_E = 320000
_D = 128
_H = 64

_NP = 10240
_NC = 2
_NS = 16
_NW = _NC * _NS
_CH = 128
_EP = 10240
_EPAD = _EP * _NW
_NCHUNK = _EP // _CH
_RD = _NP // _NS
_EB = 512
_NB = 1024


def _mesh():
    return plsc.VectorSubcoreMesh(core_axis_name="c", subcore_axis_name="s")


def _sc_params():
    return pltpu.CompilerParams(needs_layout_passes=False)


def _silu(v):
    return v * jax.nn.sigmoid(v)


def _bcast16(ref, e):
    return plsc.load_gather(ref, [jnp.zeros((16,), I32) + e])


def _worker_id():
    return lax.axis_index("c") * _NS + lax.axis_index("s")


def _sc_count(src, dst):
    def body(src_ref, dst_ref, outd_ref, outs_ref, ibuf, accd, accs):
        wid = _worker_id()
        zero16 = jnp.zeros((16,), F32)
        ones16 = jnp.ones((16,), F32)
        iota16 = lax.iota(I32, 16)

        def zb(i, _):
            accd[pl.ds(i * 16, 16)] = zero16
            accs[pl.ds(i * 16, 16)] = zero16
            return 0

        lax.fori_loop(0, _NP // 16, zb, 0)

        def chunk(ci, _):
            base = wid * _EP + ci * _CH
            pltpu.sync_copy(src_ref.at[pl.ds(base, _CH)], ibuf.at[0])
            pltpu.sync_copy(dst_ref.at[pl.ds(base, _CH)], ibuf.at[1])

            def sub(j, _):
                off = j * 16
                sidx = ibuf[0, pl.ds(off, 16)]
                didx = ibuf[1, pl.ds(off, 16)]
                val = jnp.where((base + off + iota16) < _E, ones16, 0.0)
                plsc.addupdate_scatter(accd, [didx], val)
                plsc.addupdate_scatter(accs, [sidx], val)
                return 0

            lax.fori_loop(0, _CH // 16, sub, 0)
            return 0

        lax.fori_loop(0, _NCHUNK, chunk, 0)
        pltpu.sync_copy(accd, outd_ref.at[pl.ds(wid * _NP, _NP)])
        pltpu.sync_copy(accs, outs_ref.at[pl.ds(wid * _NP, _NP)])

    fn = pl.kernel(
        body,
        out_type=(jax.ShapeDtypeStruct((_NW * _NP,), F32),
                  jax.ShapeDtypeStruct((_NW * _NP,), F32)),
        mesh=_mesh(),
        compiler_params=_sc_params(),
        scratch_types=[
            pltpu.VMEM((2, _CH), I32),
            pltpu.VMEM((_NP,), F32),
            pltpu.VMEM((_NP,), F32),
        ],
    )
    return fn(src, dst)


def _sc_gcn_scatter(hWd, srcz, dst):
    def body(hw_ref, src_ref, dst_ref, out_ref, sbuf, dbuf, rows, zbuf, acc, sem):
        c = lax.axis_index("c")
        s = lax.axis_index("s")
        wid = c * _NS + s
        zero16 = jnp.zeros((16,), F32)

        def zb(i, _):
            for k in range(_D // 16):
                zbuf[i, pl.ds(k * 16, 16)] = zero16
            return 0

        lax.fori_loop(0, _CH, zb, 0)

        def zcp(i, _):
            pltpu.sync_copy(zbuf, acc.at[pl.ds(s * _RD + i * _CH, _CH)])
            return 0

        lax.fori_loop(0, _RD // _CH, zcp, 0)
        plsc.subcore_barrier()

        def chunk(ci, _):
            base = wid * _EP + ci * _CH
            pltpu.sync_copy(src_ref.at[pl.ds(base, _CH)], sbuf.at[0])
            pltpu.sync_copy(dst_ref.at[pl.ds(base, _CH)], dbuf.at[0])
            pltpu.async_copy(hw_ref.at[sbuf.at[0]], rows, sem).wait()
            pltpu.sync_copy(rows, acc.at[dbuf.at[0]], add=True)
            return 0

        lax.fori_loop(0, _NCHUNK, chunk, 0)
        plsc.subcore_barrier()
        pltpu.sync_copy(acc.at[pl.ds(s * _RD, _RD)],
                        out_ref.at[c, pl.ds(s * _RD, _RD)])

    fn = pl.kernel(
        body,
        out_type=jax.ShapeDtypeStruct((_NC, _NP, _D), F32),
        mesh=_mesh(),
        compiler_params=_sc_params(),
        scratch_types=[
            pltpu.VMEM((1, _CH), I32),
            pltpu.VMEM((1, _CH), I32),
            pltpu.VMEM((_CH, _D), F32),
            pltpu.VMEM((_CH, _D), F32),
            pltpu.VMEM_SHARED((_NP, _D), F32),
            pltpu.SemaphoreType.DMA,
        ],
    )
    return fn(hWd, srcz, dst)


def _sc_edge_pre(T, C, src, dst):
    def body(t_ref, c_ref, src_ref, dst_ref, pre_ref, rad_ref,
             sbuf, dbuf, ts, td, cs, cd, prebuf, radbuf,
             sem0, sem1, sem2, sem3):
        wid = _worker_id()
        iota16 = lax.iota(I32, 16)

        def chunk(ci, _):
            base = wid * _EP + ci * _CH
            pltpu.sync_copy(src_ref.at[pl.ds(base, _CH)], sbuf.at[0])
            pltpu.sync_copy(dst_ref.at[pl.ds(base, _CH)], dbuf.at[0])
            c0 = pltpu.async_copy(t_ref.at[sbuf.at[0]], ts, sem0)
            c1 = pltpu.async_copy(t_ref.at[dbuf.at[0]], td, sem1)
            c2 = pltpu.async_copy(c_ref.at[sbuf.at[0]], cs, sem2)
            c3 = pltpu.async_copy(c_ref.at[dbuf.at[0]], cd, sem3)
            c0.wait()
            c1.wait()
            c2.wait()
            c3.wait()

            def sub(cc, _):
                rvec = jnp.zeros((16,), F32)
                for j in range(16):
                    e = cc * 16 + j
                    for k in range(_H // 16):
                        prebuf[e, pl.ds(k * 16, 16)] = (
                            ts[e, pl.ds(k * 16, 16)]
                            + td[e, pl.ds(_H + k * 16, 16)])
                    acc = jnp.zeros((16,), F32)
                    for k in range(_D // 16):
                        sl = pl.ds(k * 16, 16)
                        d = cs[e, sl] - cd[e, sl]
                        acc = acc + d * d
                    rvec = jnp.where(iota16 == j, jnp.sum(acc), rvec)
                radbuf[pl.ds(cc * 16, 16)] = rvec
                return 0

            lax.fori_loop(0, _CH // 16, sub, 0)
            pltpu.sync_copy(prebuf, pre_ref.at[pl.ds(base, _CH)])
            pltpu.sync_copy(radbuf, rad_ref.at[pl.ds(base, _CH)])
            return 0

        lax.fori_loop(0, _NCHUNK, chunk, 0)

    fn = pl.kernel(
        body,
        out_type=(jax.ShapeDtypeStruct((_EPAD, _H), F32),
                  jax.ShapeDtypeStruct((_EPAD,), F32)),
        mesh=_mesh(),
        compiler_params=_sc_params(),
        scratch_types=[
            pltpu.VMEM((1, _CH), I32),
            pltpu.VMEM((1, _CH), I32),
            pltpu.VMEM((_CH, _D), F32),
            pltpu.VMEM((_CH, _D), F32),
            pltpu.VMEM((_CH, _D), F32),
            pltpu.VMEM((_CH, _D), F32),
            pltpu.VMEM((_CH, _H), F32),
            pltpu.VMEM((_CH,), F32),
            pltpu.SemaphoreType.DMA,
            pltpu.SemaphoreType.DMA,
            pltpu.SemaphoreType.DMA,
            pltpu.SemaphoreType.DMA,
        ],
    )
    return fn(T, C, src, dst)


def _sc_scatter_m(mp, src):
    def body(mp_ref, src_ref, out_ref, sbuf, mbuf, zbuf, acc, sem):
        c = lax.axis_index("c")
        s = lax.axis_index("s")
        wid = c * _NS + s
        zero16 = jnp.zeros((16,), F32)

        def zb(i, _):
            for k in range(_D // 16):
                zbuf[i, pl.ds(k * 16, 16)] = zero16
            return 0

        lax.fori_loop(0, _CH, zb, 0)

        def zcp(i, _):
            pltpu.sync_copy(zbuf, acc.at[pl.ds(s * _RD + i * _CH, _CH)])
            return 0

        lax.fori_loop(0, _RD // _CH, zcp, 0)
        plsc.subcore_barrier()

        def chunk(ci, _):
            base = wid * _EP + ci * _CH
            pltpu.sync_copy(src_ref.at[pl.ds(base, _CH)], sbuf.at[0])
            pltpu.async_copy(mp_ref.at[pl.ds(base, _CH)], mbuf, sem).wait()
            pltpu.sync_copy(mbuf, acc.at[sbuf.at[0]], add=True)
            return 0

        lax.fori_loop(0, _NCHUNK, chunk, 0)
        plsc.subcore_barrier()
        pltpu.sync_copy(acc.at[pl.ds(s * _RD, _RD)],
                        out_ref.at[c, pl.ds(s * _RD, _RD)])

    fn = pl.kernel(
        body,
        out_type=jax.ShapeDtypeStruct((_NC, _NP, _D), F32),
        mesh=_mesh(),
        compiler_params=_sc_params(),
        scratch_types=[
            pltpu.VMEM((1, _CH), I32),
            pltpu.VMEM((_CH, _D), F32),
            pltpu.VMEM((_CH, _D), F32),
            pltpu.VMEM_SHARED((_NP, _D), F32),
            pltpu.SemaphoreType.DMA,
        ],
    )
    return fn(mp, src)


def _sc_scatter_p(phi, src, dst, C):
    def body(phi_ref, src_ref, dst_ref, c_ref, out_ref,
             sbuf, dbuf, pbuf, cd, zbuf, acc, sem):
        c = lax.axis_index("c")
        s = lax.axis_index("s")
        wid = c * _NS + s
        zero16 = jnp.zeros((16,), F32)

        def zb(i, _):
            for k in range(_D // 16):
                zbuf[i, pl.ds(k * 16, 16)] = zero16
            return 0

        lax.fori_loop(0, _CH, zb, 0)

        def zcp(i, _):
            pltpu.sync_copy(zbuf, acc.at[pl.ds(s * _RD + i * _CH, _CH)])
            return 0

        lax.fori_loop(0, _RD // _CH, zcp, 0)
        plsc.subcore_barrier()

        def chunk(ci, _):
            base = wid * _EP + ci * _CH
            pltpu.sync_copy(src_ref.at[pl.ds(base, _CH)], sbuf.at[0])
            pltpu.sync_copy(dst_ref.at[pl.ds(base, _CH)], dbuf.at[0])
            cp = pltpu.async_copy(c_ref.at[dbuf.at[0]], cd, sem)
            pltpu.sync_copy(phi_ref.at[pl.ds(base, _CH)], pbuf)
            cp.wait()

            def scale(e, _):
                pv = _bcast16(pbuf, e)
                for k in range(_D // 16):
                    sl = pl.ds(k * 16, 16)
                    cd[e, sl] = cd[e, sl] * pv
                return 0

            lax.fori_loop(0, _CH, scale, 0)
            pltpu.sync_copy(cd, acc.at[sbuf.at[0]], add=True)
            return 0

        lax.fori_loop(0, _NCHUNK, chunk, 0)
        plsc.subcore_barrier()
        pltpu.sync_copy(acc.at[pl.ds(s * _RD, _RD)],
                        out_ref.at[c, pl.ds(s * _RD, _RD)])

    fn = pl.kernel(
        body,
        out_type=jax.ShapeDtypeStruct((_NC, _NP, _D), F32),
        mesh=_mesh(),
        compiler_params=_sc_params(),
        scratch_types=[
            pltpu.VMEM((1, _CH), I32),
            pltpu.VMEM((1, _CH), I32),
            pltpu.VMEM((_CH,), F32),
            pltpu.VMEM((_CH, _D), F32),
            pltpu.VMEM((_CH, _D), F32),
            pltpu.VMEM_SHARED((_NP, _D), F32),
            pltpu.SemaphoreType.DMA,
        ],
    )
    return fn(phi, src, dst, C)


def _dot(a, b):
    return jnp.dot(a, b, preferred_element_type=F32)


def _row_valid(shape):
    rows = pl.program_id(0) * shape[0] + lax.broadcasted_iota(I32, shape, 0)
    return rows < _N


def _tc_gcn_first(x, W, dinv):
    def body(x_ref, w_ref, d_ref, o_ref):
        hw = _dot(x_ref[...], w_ref[...]) * d_ref[...]
        o_ref[...] = jnp.where(_row_valid((_NB, _D)),
                               jnp.concatenate([hw, jnp.zeros((_NB, _H), F32)],
                                               axis=1), 0.0)

    return pl.pallas_call(
        body,
        grid=(_NP // _NB,),
        in_specs=[pl.BlockSpec((_NB, _D), lambda i: (i, 0)),
                  pl.BlockSpec((_D, _H), lambda i: (0, 0)),
                  pl.BlockSpec((_NB, 1), lambda i: (i, 0))],
        out_specs=pl.BlockSpec((_NB, _D), lambda i: (i, 0)),
        out_shape=jax.ShapeDtypeStruct((_NP, _D), F32),
    )(x, W, dinv)


def _tc_prep(cdT, csT):
    def body(cd_ref, cs_ref, dinv_ref, cnt_ref):
        deg = jnp.sum(cd_ref[...], axis=1, keepdims=True) + 1.0
        dinv_ref[...] = lax.rsqrt(deg)
        cnt_ref[...] = jnp.maximum(jnp.sum(cs_ref[...], axis=1, keepdims=True), 1.0)

    return pl.pallas_call(
        body,
        grid=(_NP // _NB,),
        in_specs=[pl.BlockSpec((_NB, _NW), lambda i: (i, 0)),
                  pl.BlockSpec((_NB, _NW), lambda i: (i, 0))],
        out_specs=[pl.BlockSpec((_NB, 1), lambda i: (i, 0))] * 2,
        out_shape=[jax.ShapeDtypeStruct((_NP, 1), F32)] * 2,
    )(cdT, csT)


def _tc_gcn_combine(part, hWd, dinv, b, Wnext):
    def body(p_ref, hw_ref, d_ref, b_ref, w_ref, o_ref):
        pre = (p_ref[0, :, : _H] + p_ref[1, :, : _H] + hw_ref[:, : _H]) * d_ref[...]
        h = jnp.maximum(pre + b_ref[...], 0.0)
        hw = _dot(h, w_ref[...]) * d_ref[...]
        o_ref[...] = jnp.where(_row_valid((_NB, _D)),
                               jnp.concatenate([hw, jnp.zeros((_NB, _H), F32)],
                                               axis=1), 0.0)

    return pl.pallas_call(
        body,
        grid=(_NP // _NB,),
        in_specs=[pl.BlockSpec((_NC, _NB, _D), lambda i: (0, i, 0)),
                  pl.BlockSpec((_NB, _D), lambda i: (i, 0)),
                  pl.BlockSpec((_NB, 1), lambda i: (i, 0)),
                  pl.BlockSpec((1, _H), lambda i: (0, 0)),
                  pl.BlockSpec((_H, _H), lambda i: (0, 0))],
        out_specs=pl.BlockSpec((_NB, _D), lambda i: (i, 0)),
        out_shape=jax.ShapeDtypeStruct((_NP, _D), F32),
    )(part, hWd, dinv, b, Wnext)


def _tc_gcn_final(part, hWd, dinv, b, WA, WB):
    def body(p_ref, hw_ref, d_ref, b_ref, wa_ref, wb_ref, h_ref, t_ref):
        pre = (p_ref[0, :, : _H] + p_ref[1, :, : _H] + hw_ref[:, : _H]) * d_ref[...]
        h = pre + b_ref[...]
        h_ref[...] = h
        t_ref[...] = jnp.concatenate(
            [_dot(h, wa_ref[...]), _dot(h, wb_ref[...])], axis=1)

    return pl.pallas_call(
        body,
        grid=(_NP // _NB,),
        in_specs=[pl.BlockSpec((_NC, _NB, _D), lambda i: (0, i, 0)),
                  pl.BlockSpec((_NB, _D), lambda i: (i, 0)),
                  pl.BlockSpec((_NB, 1), lambda i: (i, 0)),
                  pl.BlockSpec((1, _H), lambda i: (0, 0)),
                  pl.BlockSpec((_H, _H), lambda i: (0, 0)),
                  pl.BlockSpec((_H, _H), lambda i: (0, 0))],
        out_specs=[pl.BlockSpec((_NB, _H), lambda i: (i, 0)),
                   pl.BlockSpec((_NB, _D), lambda i: (i, 0))],
        out_shape=[jax.ShapeDtypeStruct((_NP, _H), F32),
                   jax.ShapeDtypeStruct((_NP, _D), F32)],
    )(part, hWd, dinv, b, WA, WB)


def _tc_edge_mlp(pre, rad, ea, wr, wea, eb1, eW2, eb2, cW1, cb1, cW2):
    def body(pre_ref, rad_ref, ea_ref, wr_ref, wea_ref, eb1_ref,
             ew2_ref, eb2_ref, cw1_ref, cb1_ref, cw2_ref, m_ref, phi_ref):
        eab = ea_ref[...]
        t = (pre_ref[...] + rad_ref[...] * wr_ref[...]
             + eab[:, 0:1] * wea_ref[0:1, :] + eab[:, 1:2] * wea_ref[1:2, :]
             + eb1_ref[...])
        m = _silu(_dot(_silu(t), ew2_ref[...]) + eb2_ref[...])
        phi = _dot(_silu(_dot(m, cw1_ref[...]) + cb1_ref[...]), cw2_ref[...])
        valid = (pl.program_id(0) * _EB
                 + lax.broadcasted_iota(I32, (_EB, 1), 0)) < _E
        phiz = jnp.where(valid, phi, 0.0)
        m_ref[...] = jnp.where(
            valid, jnp.concatenate(
                [m, phi, jnp.zeros((_EB, _D - _H - 1), F32)], axis=1), 0.0)
        phi_ref[...] = phiz

    return pl.pallas_call(
        body,
        grid=(_EPAD // _EB,),
        in_specs=[pl.BlockSpec((_EB, _H), lambda i: (i, 0)),
                  pl.BlockSpec((_EB, 1), lambda i: (i, 0)),
                  pl.BlockSpec((_EB, 2), lambda i: (i, 0)),
                  pl.BlockSpec((1, _H), lambda i: (0, 0)),
                  pl.BlockSpec((2, _H), lambda i: (0, 0)),
                  pl.BlockSpec((1, _H), lambda i: (0, 0)),
                  pl.BlockSpec((_H, _H), lambda i: (0, 0)),
                  pl.BlockSpec((1, _H), lambda i: (0, 0)),
                  pl.BlockSpec((_H, _H), lambda i: (0, 0)),
                  pl.BlockSpec((1, _H), lambda i: (0, 0)),
                  pl.BlockSpec((_H, 1), lambda i: (0, 0))],
        out_specs=[pl.BlockSpec((_EB, _D), lambda i: (i, 0)),
                   pl.BlockSpec((_EB, 1), lambda i: (i, 0))],
        out_shape=[jax.ShapeDtypeStruct((_EPAD, _D), F32),
                   jax.ShapeDtypeStruct((_EPAD, 1), F32)],
    )(pre, rad, ea, wr, wea, eb1, eW2, eb2, cW1, cb1, cW2)


def _tc_node_full(h, mpp, pp, coord, cnt, w1a, w1b, b1, w2, b2, WA, WB):
    def body(h_ref, mpp_ref, pp_ref, co_ref, cnt_ref,
             w1a_ref, w1b_ref, b1_ref, w2_ref, b2_ref, wa_ref, wb_ref,
             hn_ref, con_ref, t_ref):
        h = h_ref[...]
        mps = mpp_ref[0] + mpp_ref[1]
        agg = mps[:, : _H]
        S = mps[:, _H: _H + 1]
        u = _dot(_silu(_dot(h, w1a_ref[...]) + _dot(agg, w1b_ref[...])
                       + b1_ref[...]), w2_ref[...]) + b2_ref[...]
        hn = h + u
        hn_ref[...] = hn
        P = pp_ref[0] + pp_ref[1]
        co = co_ref[...]
        con_ref[...] = co + (co * S - P) / cnt_ref[...]
        t_ref[...] = jnp.concatenate(
            [_dot(hn, wa_ref[...]), _dot(hn, wb_ref[...])], axis=1)

    return pl.pallas_call(
        body,
        grid=(_NP // _NB,),
        in_specs=[pl.BlockSpec((_NB, _H), lambda i: (i, 0)),
                  pl.BlockSpec((_NC, _NB, _D), lambda i: (0, i, 0)),
                  pl.BlockSpec((_NC, _NB, _D), lambda i: (0, i, 0)),
                  pl.BlockSpec((_NB, _D), lambda i: (i, 0)),
                  pl.BlockSpec((_NB, 1), lambda i: (i, 0)),
                  pl.BlockSpec((_H, _H), lambda i: (0, 0)),
                  pl.BlockSpec((_H, _H), lambda i: (0, 0)),
                  pl.BlockSpec((1, _H), lambda i: (0, 0)),
                  pl.BlockSpec((_H, _H), lambda i: (0, 0)),
                  pl.BlockSpec((1, _H), lambda i: (0, 0)),
                  pl.BlockSpec((_H, _H), lambda i: (0, 0)),
                  pl.BlockSpec((_H, _H), lambda i: (0, 0))],
        out_specs=[pl.BlockSpec((_NB, _H), lambda i: (i, 0)),
                   pl.BlockSpec((_NB, _D), lambda i: (i, 0)),
                   pl.BlockSpec((_NB, _D), lambda i: (i, 0))],
        out_shape=[jax.ShapeDtypeStruct((_NP, _H), F32),
                   jax.ShapeDtypeStruct((_NP, _D), F32),
                   jax.ShapeDtypeStruct((_NP, _D), F32)],
    )(h, mpp, pp, coord, cnt, w1a, w1b, b1, w2, b2, WA, WB)


def _tc_node_last(h, mpp, w1a, w1b, b1, w2, b2):
    def body(h_ref, mpp_ref, w1a_ref, w1b_ref, b1_ref, w2_ref, b2_ref, hn_ref):
        h = h_ref[...]
        agg = (mpp_ref[0] + mpp_ref[1])[:, : _H]
        u = _dot(_silu(_dot(h, w1a_ref[...]) + _dot(agg, w1b_ref[...])
                       + b1_ref[...]), w2_ref[...]) + b2_ref[...]
        hn_ref[...] = h + u

    return pl.pallas_call(
        body,
        grid=(_NP // _NB,),
        in_specs=[pl.BlockSpec((_NB, _H), lambda i: (i, 0)),
                  pl.BlockSpec((_NC, _NB, _D), lambda i: (0, i, 0)),
                  pl.BlockSpec((_H, _H), lambda i: (0, 0)),
                  pl.BlockSpec((_H, _H), lambda i: (0, 0)),
                  pl.BlockSpec((1, _H), lambda i: (0, 0)),
                  pl.BlockSpec((_H, _H), lambda i: (0, 0)),
                  pl.BlockSpec((1, _H), lambda i: (0, 0))],
        out_specs=pl.BlockSpec((_NB, _H), lambda i: (i, 0)),
        out_shape=jax.ShapeDtypeStruct((_NP, _H), F32),
    )(h, mpp, w1a, w1b, b1, w2, b2)


def kernel(x, edge_index, edge_attr, params):
    src = edge_index[0].astype(I32)
    dst = edge_index[1].astype(I32)
    pad = _EPAD - _E
    srcp = jnp.concatenate([src, jnp.zeros((pad,), I32)])
    srcz = jnp.concatenate([src, jnp.full((pad,), _N, I32)])
    dstp = jnp.concatenate([dst, jnp.zeros((pad,), I32)])
    eap = jnp.concatenate([edge_attr.astype(F32), jnp.zeros((pad, 2), F32)])
    xp = jnp.concatenate([x.astype(F32), jnp.zeros((_NP - _N, _D), F32)])

    cd_flat, cs_flat = _sc_count(srcp, dstp)
    cdT = cd_flat.reshape(_NW, _NP).T
    csT = cs_flat.reshape(_NW, _NP).T
    dinv_c, cnt_c = _tc_prep(cdT, csT)

    gcn = params["gcn"]
    egcl = params["egcl"]
    hWd = _tc_gcn_first(xp, gcn[0][0], dinv_c)
    for l in range(len(gcn)):
        part = _sc_gcn_scatter(hWd, srcz, dstp)
        bias = gcn[l][1].reshape(1, _H)
        if l < len(gcn) - 1:
            hWd = _tc_gcn_combine(part, hWd, dinv_c, bias, gcn[l + 1][0])
        else:
            e0 = egcl[0]
            h, T = _tc_gcn_final(part, hWd, dinv_c, bias,
                                 e0["eW1"][:_H], e0["eW1"][_H:2 * _H])

    coord = xp
    for l, p in enumerate(egcl):
        pre, rad = _sc_edge_pre(T, coord, srcp, dstp)
        mp, phi = _tc_edge_mlp(
            pre, rad.reshape(_EPAD, 1), eap,
            p["eW1"][2 * _H:2 * _H + 1], p["eW1"][2 * _H + 1:],
            p["eb1"].reshape(1, _H), p["eW2"], p["eb2"].reshape(1, _H),
            p["cW1"], p["cb1"].reshape(1, _H), p["cW2"])
        phi1 = phi.reshape(_EPAD)
        mpp = _sc_scatter_m(mp, srcp)
        if l < len(egcl) - 1:
            pp = _sc_scatter_p(phi1, srcp, dstp, coord)
            pn = egcl[l + 1]
            h, coord, T = _tc_node_full(
                h, mpp, pp, coord, cnt_c,
                p["nW1"][:_H], p["nW1"][_H:], p["nb1"].reshape(1, _H),
                p["nW2"], p["nb2"].reshape(1, _H),
                pn["eW1"][:_H], pn["eW1"][_H:2 * _H])
        else:
            h = _tc_node_last(
                h, mpp,
                p["nW1"][:_H], p["nW1"][_H:], p["nb1"].reshape(1, _H),
                p["nW2"], p["nb2"].reshape(1, _H))
    return h[:_N]


# speedup vs baseline: 2.1967x; 1.1908x over previous
import jax
import jax.numpy as jnp
from jax import lax
from jax.experimental import pallas as pl
from jax.experimental.pallas import tpu as pltpu
from jax.experimental.pallas import tpu_sc as plsc

F32 = jnp.float32
I32 = jnp.int32

_N = 10000
_E = 320000
_D = 128
_H = 64
_TW = 256

_NP = 10240
_NC = 2
_NS = 16
_NW = _NC * _NS
_CH = 128
_EP = 10240
_EPAD = _EP * _NW
_NCHUNK = _EP // _CH
_CH2 = 64
_NCHUNK2 = _EP // _CH2
_RD = _NP // _NS
_EB = 512
_NB = 1024


def _mesh():
    return plsc.VectorSubcoreMesh(core_axis_name="c", subcore_axis_name="s")


def _sc_params():
    return pltpu.CompilerParams(needs_layout_passes=False)


def _silu(v):
    return v * jax.nn.sigmoid(v)


def _worker_id():
    return lax.axis_index("c") * _NS + lax.axis_index("s")


def _sc_count(src2, dst2):
    def body(src_ref, dst_ref, outd_ref, outs_ref, sbuf, dbuf, accd, accs):
        wid = _worker_id()
        zero16 = jnp.zeros((16,), F32)
        ones16 = jnp.ones((16,), F32)
        iota16 = lax.iota(I32, 16)

        def zb(i, _):
            accd[pl.ds(i * 16, 16)] = zero16
            accs[pl.ds(i * 16, 16)] = zero16
            return 0

        lax.fori_loop(0, _NP // 16, zb, 0)
        pltpu.sync_copy(src_ref.at[pl.ds(wid * _NCHUNK, _NCHUNK)], sbuf)
        pltpu.sync_copy(dst_ref.at[pl.ds(wid * _NCHUNK, _NCHUNK)], dbuf)

        def sub(t, _):
            ci = t // (_CH // 16)
            off = (t % (_CH // 16)) * 16
            sidx = sbuf[ci, pl.ds(off, 16)]
            didx = dbuf[ci, pl.ds(off, 16)]
            gid = wid * _EP + ci * _CH + off + iota16
            val = jnp.where(gid < _E, ones16, 0.0)
            plsc.addupdate_scatter(accd, [didx], val)
            plsc.addupdate_scatter(accs, [sidx], val)
            return 0

        lax.fori_loop(0, _NCHUNK * (_CH // 16), sub, 0)
        pltpu.sync_copy(accd, outd_ref.at[pl.ds(wid * _NP, _NP)])
        pltpu.sync_copy(accs, outs_ref.at[pl.ds(wid * _NP, _NP)])

    fn = pl.kernel(
        body,
        out_type=(jax.ShapeDtypeStruct((_NW * _NP,), F32),
                  jax.ShapeDtypeStruct((_NW * _NP,), F32)),
        mesh=_mesh(),
        compiler_params=_sc_params(),
        scratch_types=[
            pltpu.VMEM((_NCHUNK, _CH), I32),
            pltpu.VMEM((_NCHUNK, _CH), I32),
            pltpu.VMEM((_NP,), F32),
            pltpu.VMEM((_NP,), F32),
        ],
    )
    return fn(src2, dst2)


def _zero_acc_rows(row0, acc, s, width, ch):
    zero16 = jnp.zeros((16,), F32)

    def zb(i, _):
        for k in range(width // 16):
            row0[i, pl.ds(k * 16, 16)] = zero16
        return 0

    lax.fori_loop(0, ch, zb, 0)

    def zcp(i, _):
        pltpu.sync_copy(row0, acc.at[pl.ds(s * _RD + i * ch, ch)])
        return 0

    lax.fori_loop(0, _RD // ch, zcp, 0)


def _gather_scatter_body(gather_src, sbuf_scatter, out_ref, rows, gsem, ssem,
                         acc, c, s, nchunk, la):
    K = len(rows)
    gd = [None] * nchunk
    sd = [None] * nchunk
    for ci in range(nchunk + la):
        if ci < nchunk:
            b = ci % K
            if ci >= K:
                sd[ci - K].wait()
            gd[ci] = pltpu.async_copy(gather_src(ci), rows[b], gsem[b])
        cj = ci - la
        if cj >= 0:
            b2 = cj % K
            gd[cj].wait()
            sd[cj] = pltpu.async_copy(
                rows[b2], acc.at[sbuf_scatter.at[cj, 0]], ssem[b2], add=True)
    for cj in range(nchunk - K, nchunk):
        sd[cj].wait()
    plsc.subcore_barrier()
    pltpu.sync_copy(acc.at[pl.ds(s * _RD, _RD)],
                    out_ref.at[c, pl.ds(s * _RD, _RD)])


_NCH = _NCHUNK2 // 2


def _sc_gcn_scatter(hWd, srcz64, dst64, off):
    def body(hw_ref, src_ref, dst_ref, out_ref,
             sbuf, dbuf, r0, r1, r2, acc, g0, g1, g2, s0, s1, s2):
        c = lax.axis_index("c")
        s = lax.axis_index("s")
        wid = c * _NS + s
        _zero_acc_rows(r0, acc, s, _D, _CH2)
        pltpu.sync_copy(src_ref.at[pl.ds(wid * _NCHUNK2 + off, _NCH)], sbuf)
        pltpu.sync_copy(dst_ref.at[pl.ds(wid * _NCHUNK2 + off, _NCH)], dbuf)
        plsc.subcore_barrier()
        _gather_scatter_body(
            lambda ci: hw_ref.at[sbuf.at[ci, 0]], dbuf, out_ref,
            [r0, r1, r2], [g0, g1, g2], [s0, s1, s2], acc, c, s, _NCH, 2)

    fn = pl.kernel(
        body,
        out_type=jax.ShapeDtypeStruct((_NC, _NP, _D), F32),
        mesh=_mesh(),
        compiler_params=_sc_params(),
        scratch_types=[
            pltpu.VMEM((_NCH, 1, _CH2), I32),
            pltpu.VMEM((_NCH, 1, _CH2), I32),
            pltpu.VMEM((_CH2, _D), F32),
            pltpu.VMEM((_CH2, _D), F32),
            pltpu.VMEM((_CH2, _D), F32),
            pltpu.VMEM_SHARED((_NP, _D), F32),
        ] + [pltpu.SemaphoreType.DMA] * 6,
    )
    return fn(hWd, srcz64, dst64)


def _sc_scatter_m(mp, src3):
    def body(mp_ref, src_ref, out_ref, sbuf, r0, r1, acc, g0, g1, s0, s1):
        c = lax.axis_index("c")
        s = lax.axis_index("s")
        wid = c * _NS + s
        _zero_acc_rows(r0, acc, s, _D, _CH)
        pltpu.sync_copy(src_ref.at[pl.ds(wid * _NCHUNK, _NCHUNK)], sbuf)
        plsc.subcore_barrier()
        _gather_scatter_body(
            lambda ci: mp_ref.at[pl.ds(wid * _EP + ci * _CH, _CH)], sbuf,
            out_ref, [r0, r1], [g0, g1], [s0, s1], acc, c, s, _NCHUNK, 1)

    fn = pl.kernel(
        body,
        out_type=jax.ShapeDtypeStruct((_NC, _NP, _D), F32),
        mesh=_mesh(),
        compiler_params=_sc_params(),
        scratch_types=[
            pltpu.VMEM((_NCHUNK, 1, _CH), I32),
            pltpu.VMEM((_CH, _D), F32),
            pltpu.VMEM((_CH, _D), F32),
            pltpu.VMEM_SHARED((_NP, _D), F32),
        ] + [pltpu.SemaphoreType.DMA] * 4,
    )
    return fn(mp, src3)


def _sc_edge_pre(TC3, src64, dst64):
    def body(t_ref, src_ref, dst_ref, pre_ref, rad_ref,
             sbuf, dbuf, ts0, ts1, td0, td1, prebuf, radbuf,
             gs0, gs1, gd0, gd1):
        wid = _worker_id()
        iota16 = lax.iota(I32, 16)
        ts = [ts0, ts1]
        td = [td0, td1]
        gs = [gs0, gs1]
        gdm = [gd0, gd1]
        pltpu.sync_copy(src_ref.at[pl.ds(wid * _NCHUNK2, _NCHUNK2)], sbuf)
        pltpu.sync_copy(dst_ref.at[pl.ds(wid * _NCHUNK2, _NCHUNK2)], dbuf)
        for b in range(2):
            pltpu.async_copy(t_ref.at[sbuf.at[b, 0]], ts[b], gs[b])
            pltpu.async_copy(t_ref.at[dbuf.at[b, 0]], td[b], gdm[b])

        def outer(g, _):
            for b in range(2):
                ci = g * 2 + b
                pltpu.make_async_copy(t_ref.at[sbuf.at[0, 0]], ts[b], gs[b]).wait()
                pltpu.make_async_copy(t_ref.at[dbuf.at[0, 0]], td[b], gdm[b]).wait()

                def sub(cc, _):
                    rvec = jnp.zeros((16,), F32)
                    for j in range(16):
                        e = cc * 16 + j
                        for k in range(_H // 16):
                            prebuf[e, pl.ds(k * 16, 16)] = (
                                ts[b][e, pl.ds(k * 16, 16)]
                                + td[b][e, pl.ds(_H + k * 16, 16)])
                        acc = jnp.zeros((16,), F32)
                        for k in range(_D // 16):
                            sl = pl.ds(2 * _H + k * 16, 16)
                            d = ts[b][e, sl] - td[b][e, sl]
                            acc = acc + d * d
                        rvec = jnp.where(iota16 == j, jnp.sum(acc), rvec)
                    radbuf[pl.ds(b * _CH2 + cc * 16, 16)] = rvec
                    return 0

                lax.fori_loop(0, _CH2 // 16, sub, 0)
                pltpu.sync_copy(prebuf,
                                pre_ref.at[pl.ds(wid * _EP + ci * _CH2, _CH2)])
                nxt = jnp.minimum(ci + 2, _NCHUNK2 - 1)
                pltpu.async_copy(t_ref.at[sbuf.at[nxt, 0]], ts[b], gs[b])
                pltpu.async_copy(t_ref.at[dbuf.at[nxt, 0]], td[b], gdm[b])
            pltpu.sync_copy(radbuf,
                            rad_ref.at[pl.ds(wid * _EP + g * 2 * _CH2,
                                             2 * _CH2)])
            return 0

        lax.fori_loop(0, _NCHUNK2 // 2, outer, 0)
        for b in range(2):
            pltpu.make_async_copy(t_ref.at[sbuf.at[0, 0]], ts[b], gs[b]).wait()
            pltpu.make_async_copy(t_ref.at[dbuf.at[0, 0]], td[b], gdm[b]).wait()

    fn = pl.kernel(
        body,
        out_type=(jax.ShapeDtypeStruct((_EPAD, _H), F32),
                  jax.ShapeDtypeStruct((_EPAD,), F32)),
        mesh=_mesh(),
        compiler_params=_sc_params(),
        scratch_types=[
            pltpu.VMEM((_NCHUNK2, 1, _CH2), I32),
            pltpu.VMEM((_NCHUNK2, 1, _CH2), I32),
            pltpu.VMEM((_CH2, _TW), F32),
            pltpu.VMEM((_CH2, _TW), F32),
            pltpu.VMEM((_CH2, _TW), F32),
            pltpu.VMEM((_CH2, _TW), F32),
            pltpu.VMEM((_CH2, _H), F32),
            pltpu.VMEM((2 * _CH2,), F32),
        ] + [pltpu.SemaphoreType.DMA] * 4,
    )
    return fn(TC3, src64, dst64)


def _sc_scatter_p(phi2, src3, dst3, C, off):
    def body(phi_ref, src_ref, dst_ref, c_ref, out_ref,
             sbuf, dbuf, pbuf, r0, r1, acc, g0, g1, s0, s1):
        c = lax.axis_index("c")
        s = lax.axis_index("s")
        wid = c * _NS + s
        rows = [r0, r1]
        gsem = [g0, g1]
        ssem = [s0, s1]
        _zero_acc_rows(r0, acc, s, _D, _CH2)
        pltpu.sync_copy(src_ref.at[pl.ds(wid * _NCHUNK2 + off, _NCH)], sbuf)
        pltpu.sync_copy(dst_ref.at[pl.ds(wid * _NCHUNK2 + off, _NCH)], dbuf)
        pltpu.sync_copy(phi_ref.at[pl.ds(wid * _NCHUNK2 + off, _NCH)], pbuf)
        plsc.subcore_barrier()
        for b in range(2):
            pltpu.async_copy(c_ref.at[dbuf.at[b, 0]], rows[b], gsem[b])

        def outer(g, _):
            for b in range(2):
                ci = g * 2 + b
                pltpu.make_async_copy(c_ref.at[dbuf.at[0, 0]], rows[b],
                                      gsem[b]).wait()

                def scale(e, _):
                    pv = plsc.load_gather(
                        pbuf, [jnp.zeros((16,), I32) + ci,
                               jnp.zeros((16,), I32) + e])
                    for k in range(_D // 16):
                        sl = pl.ds(k * 16, 16)
                        rows[b][e, sl] = rows[b][e, sl] * pv
                    return 0

                lax.fori_loop(0, _CH2, scale, 0)
                pltpu.async_copy(rows[b], acc.at[sbuf.at[ci, 0]], ssem[b],
                                 add=True).wait()
                nxt = jnp.minimum(ci + 2, _NCH - 1)
                pltpu.async_copy(c_ref.at[dbuf.at[nxt, 0]], rows[b], gsem[b])
            return 0

        lax.fori_loop(0, _NCH // 2, outer, 0)
        for b in range(2):
            pltpu.make_async_copy(c_ref.at[dbuf.at[0, 0]], rows[b],
                                  gsem[b]).wait()
        plsc.subcore_barrier()
        pltpu.sync_copy(acc.at[pl.ds(s * _RD, _RD)],
                        out_ref.at[c, pl.ds(s * _RD, _RD)])

    fn = pl.kernel(
        body,
        out_type=jax.ShapeDtypeStruct((_NC, _NP, _D), F32),
        mesh=_mesh(),
        compiler_params=_sc_params(),
        scratch_types=[
            pltpu.VMEM((_NCH, 1, _CH2), I32),
            pltpu.VMEM((_NCH, 1, _CH2), I32),
            pltpu.VMEM((_NCH, _CH2), F32),
            pltpu.VMEM((_CH2, _D), F32),
            pltpu.VMEM((_CH2, _D), F32),
            pltpu.VMEM_SHARED((_NP, _D), F32),
        ] + [pltpu.SemaphoreType.DMA] * 4,
    )
    return fn(phi2, src3, dst3, C)


def _dot(a, b):
    return jnp.dot(a, b, preferred_element_type=F32)


def _row_valid(shape):
    rows = pl.program_id(0) * shape[0] + lax.broadcasted_iota(I32, shape, 0)
    return rows < _N


def _tc_gcn_first(x, W, dinv):
    def body(x_ref, w_ref, d_ref, o_ref):
        hw = _dot(x_ref[...], w_ref[...]) * d_ref[...]
        o_ref[...] = jnp.where(_row_valid((_NB, _D)),
                               jnp.concatenate([hw, jnp.zeros((_NB, _H), F32)],
                                               axis=1), 0.0)

    return pl.pallas_call(
        body,
        grid=(_NP // _NB,),
        in_specs=[pl.BlockSpec((_NB, _D), lambda i: (i, 0)),
                  pl.BlockSpec((_D, _H), lambda i: (0, 0)),
                  pl.BlockSpec((_NB, 1), lambda i: (i, 0))],
        out_specs=pl.BlockSpec((_NB, _D), lambda i: (i, 0)),
        out_shape=jax.ShapeDtypeStruct((_NP, _D), F32),
    )(x, W, dinv)


def _tc_prep(cdT, csT):
    def body(cd_ref, cs_ref, dinv_ref, cnt_ref):
        deg = jnp.sum(cd_ref[...], axis=1, keepdims=True) + 1.0
        dinv_ref[...] = lax.rsqrt(deg)
        cnt_ref[...] = jnp.maximum(jnp.sum(cs_ref[...], axis=1, keepdims=True), 1.0)

    return pl.pallas_call(
        body,
        grid=(_NP // _NB,),
        in_specs=[pl.BlockSpec((_NB, _NW), lambda i: (i, 0)),
                  pl.BlockSpec((_NB, _NW), lambda i: (i, 0))],
        out_specs=[pl.BlockSpec((_NB, 1), lambda i: (i, 0))] * 2,
        out_shape=[jax.ShapeDtypeStruct((_NP, 1), F32)] * 2,
    )(cdT, csT)


def _tc_gcn_combine(part, part2, hWd, dinv, b, Wnext):
    def body(p_ref, q_ref, hw_ref, d_ref, b_ref, w_ref, o_ref):
        pre = (p_ref[0, :, : _H] + p_ref[1, :, : _H]
               + q_ref[0, :, : _H] + q_ref[1, :, : _H]
               + hw_ref[:, : _H]) * d_ref[...]
        h = jnp.maximum(pre + b_ref[...], 0.0)
        hw = _dot(h, w_ref[...]) * d_ref[...]
        o_ref[...] = jnp.where(_row_valid((_NB, _D)),
                               jnp.concatenate([hw, jnp.zeros((_NB, _H), F32)],
                                               axis=1), 0.0)

    return pl.pallas_call(
        body,
        grid=(_NP // _NB,),
        in_specs=[pl.BlockSpec((_NC, _NB, _D), lambda i: (0, i, 0)),
                  pl.BlockSpec((_NC, _NB, _D), lambda i: (0, i, 0)),
                  pl.BlockSpec((_NB, _D), lambda i: (i, 0)),
                  pl.BlockSpec((_NB, 1), lambda i: (i, 0)),
                  pl.BlockSpec((1, _H), lambda i: (0, 0)),
                  pl.BlockSpec((_H, _H), lambda i: (0, 0))],
        out_specs=pl.BlockSpec((_NB, _D), lambda i: (i, 0)),
        out_shape=jax.ShapeDtypeStruct((_NP, _D), F32),
    )(part, part2, hWd, dinv, b, Wnext)


def _tc_gcn_final(part, part2, hWd, dinv, b, WA, WB, coord):
    def body(p_ref, q_ref, hw_ref, d_ref, b_ref, wa_ref, wb_ref, co_ref,
             h_ref, t_ref):
        pre = (p_ref[0, :, : _H] + p_ref[1, :, : _H]
               + q_ref[0, :, : _H] + q_ref[1, :, : _H]
               + hw_ref[:, : _H]) * d_ref[...]
        h = pre + b_ref[...]
        h_ref[...] = h
        t_ref[...] = jnp.concatenate(
            [_dot(h, wa_ref[...]), _dot(h, wb_ref[...]), co_ref[...]], axis=1)

    return pl.pallas_call(
        body,
        grid=(_NP // _NB,),
        in_specs=[pl.BlockSpec((_NC, _NB, _D), lambda i: (0, i, 0)),
                  pl.BlockSpec((_NC, _NB, _D), lambda i: (0, i, 0)),
                  pl.BlockSpec((_NB, _D), lambda i: (i, 0)),
                  pl.BlockSpec((_NB, 1), lambda i: (i, 0)),
                  pl.BlockSpec((1, _H), lambda i: (0, 0)),
                  pl.BlockSpec((_H, _H), lambda i: (0, 0)),
                  pl.BlockSpec((_H, _H), lambda i: (0, 0)),
                  pl.BlockSpec((_NB, _D), lambda i: (i, 0))],
        out_specs=[pl.BlockSpec((_NB, _H), lambda i: (i, 0)),
                   pl.BlockSpec((_NB, _TW), lambda i: (i, 0))],
        out_shape=[jax.ShapeDtypeStruct((_NP, _H), F32),
                   jax.ShapeDtypeStruct((_NP, _TW), F32)],
    )(part, part2, hWd, dinv, b, WA, WB, coord)


def _tc_edge_mlp(pre, rad, ea, wr, wea, eb1, eW2, eb2, cW1, cb1, cW2):
    def body(pre_ref, rad_ref, ea_ref, wr_ref, wea_ref, eb1_ref,
             ew2_ref, eb2_ref, cw1_ref, cb1_ref, cw2_ref, m_ref, phi_ref):
        eab = ea_ref[...]
        t = (pre_ref[...] + rad_ref[...] * wr_ref[...]
             + eab[:, 0:1] * wea_ref[0:1, :] + eab[:, 1:2] * wea_ref[1:2, :]
             + eb1_ref[...])
        m = _silu(_dot(_silu(t), ew2_ref[...]) + eb2_ref[...])
        phi = _dot(_silu(_dot(m, cw1_ref[...]) + cb1_ref[...]), cw2_ref[...])
        valid = (pl.program_id(0) * _EB
                 + lax.broadcasted_iota(I32, (_EB, 1), 0)) < _E
        phiz = jnp.where(valid, phi, 0.0)
        m_ref[...] = jnp.where(
            valid, jnp.concatenate(
                [m, phi, jnp.zeros((_EB, _D - _H - 1), F32)], axis=1), 0.0)
        phi_ref[...] = phiz

    return pl.pallas_call(
        body,
        grid=(_EPAD // _EB,),
        in_specs=[pl.BlockSpec((_EB, _H), lambda i: (i, 0)),
                  pl.BlockSpec((_EB, 1), lambda i: (i, 0)),
                  pl.BlockSpec((_EB, 2), lambda i: (i, 0)),
                  pl.BlockSpec((1, _H), lambda i: (0, 0)),
                  pl.BlockSpec((2, _H), lambda i: (0, 0)),
                  pl.BlockSpec((1, _H), lambda i: (0, 0)),
                  pl.BlockSpec((_H, _H), lambda i: (0, 0)),
                  pl.BlockSpec((1, _H), lambda i: (0, 0)),
                  pl.BlockSpec((_H, _H), lambda i: (0, 0)),
                  pl.BlockSpec((1, _H), lambda i: (0, 0)),
                  pl.BlockSpec((_H, 1), lambda i: (0, 0))],
        out_specs=[pl.BlockSpec((_EB, _D), lambda i: (i, 0)),
                   pl.BlockSpec((_EB, 1), lambda i: (i, 0))],
        out_shape=[jax.ShapeDtypeStruct((_EPAD, _D), F32),
                   jax.ShapeDtypeStruct((_EPAD, 1), F32)],
    )(pre, rad, ea, wr, wea, eb1, eW2, eb2, cW1, cb1, cW2)


def _tc_node_full(h, mpp, pp, pp2, coord, cnt, w1a, w1b, b1, w2, b2, WA, WB):
    def body(h_ref, mpp_ref, pp_ref, pq_ref, co_ref, cnt_ref,
             w1a_ref, w1b_ref, b1_ref, w2_ref, b2_ref, wa_ref, wb_ref,
             hn_ref, con_ref, t_ref):
        h = h_ref[...]
        mps = mpp_ref[0] + mpp_ref[1]
        agg = mps[:, : _H]
        S = mps[:, _H: _H + 1]
        u = _dot(_silu(_dot(h, w1a_ref[...]) + _dot(agg, w1b_ref[...])
                       + b1_ref[...]), w2_ref[...]) + b2_ref[...]
        hn = h + u
        hn_ref[...] = hn
        P = pp_ref[0] + pp_ref[1] + pq_ref[0] + pq_ref[1]
        co = co_ref[...]
        con = co + (co * S - P) / cnt_ref[...]
        con_ref[...] = con
        t_ref[...] = jnp.concatenate(
            [_dot(hn, wa_ref[...]), _dot(hn, wb_ref[...]), con], axis=1)

    return pl.pallas_call(
        body,
        grid=(_NP // _NB,),
        in_specs=[pl.BlockSpec((_NB, _H), lambda i: (i, 0)),
                  pl.BlockSpec((_NC, _NB, _D), lambda i: (0, i, 0)),
                  pl.BlockSpec((_NC, _NB, _D), lambda i: (0, i, 0)),
                  pl.BlockSpec((_NC, _NB, _D), lambda i: (0, i, 0)),
                  pl.BlockSpec((_NB, _D), lambda i: (i, 0)),
                  pl.BlockSpec((_NB, 1), lambda i: (i, 0)),
                  pl.BlockSpec((_H, _H), lambda i: (0, 0)),
                  pl.BlockSpec((_H, _H), lambda i: (0, 0)),
                  pl.BlockSpec((1, _H), lambda i: (0, 0)),
                  pl.BlockSpec((_H, _H), lambda i: (0, 0)),
                  pl.BlockSpec((1, _H), lambda i: (0, 0)),
                  pl.BlockSpec((_H, _H), lambda i: (0, 0)),
                  pl.BlockSpec((_H, _H), lambda i: (0, 0))],
        out_specs=[pl.BlockSpec((_NB, _H), lambda i: (i, 0)),
                   pl.BlockSpec((_NB, _D), lambda i: (i, 0)),
                   pl.BlockSpec((_NB, _TW), lambda i: (i, 0))],
        out_shape=[jax.ShapeDtypeStruct((_NP, _H), F32),
                   jax.ShapeDtypeStruct((_NP, _D), F32),
                   jax.ShapeDtypeStruct((_NP, _TW), F32)],
    )(h, mpp, pp, pp2, coord, cnt, w1a, w1b, b1, w2, b2, WA, WB)


def _tc_node_last(h, mpp, w1a, w1b, b1, w2, b2):
    def body(h_ref, mpp_ref, w1a_ref, w1b_ref, b1_ref, w2_ref, b2_ref, hn_ref):
        h = h_ref[...]
        agg = (mpp_ref[0] + mpp_ref[1])[:, : _H]
        u = _dot(_silu(_dot(h, w1a_ref[...]) + _dot(agg, w1b_ref[...])
                       + b1_ref[...]), w2_ref[...]) + b2_ref[...]
        hn_ref[...] = h + u

    return pl.pallas_call(
        body,
        grid=(_NP // _NB,),
        in_specs=[pl.BlockSpec((_NB, _H), lambda i: (i, 0)),
                  pl.BlockSpec((_NC, _NB, _D), lambda i: (0, i, 0)),
                  pl.BlockSpec((_H, _H), lambda i: (0, 0)),
                  pl.BlockSpec((_H, _H), lambda i: (0, 0)),
                  pl.BlockSpec((1, _H), lambda i: (0, 0)),
                  pl.BlockSpec((_H, _H), lambda i: (0, 0)),
                  pl.BlockSpec((1, _H), lambda i: (0, 0))],
        out_specs=pl.BlockSpec((_NB, _H), lambda i: (i, 0)),
        out_shape=jax.ShapeDtypeStruct((_NP, _H), F32),
    )(h, mpp, w1a, w1b, b1, w2, b2)


def kernel(x, edge_index, edge_attr, params):
    src = edge_index[0].astype(I32)
    dst = edge_index[1].astype(I32)
    pad = _EPAD - _E
    srcp = jnp.concatenate([src, jnp.zeros((pad,), I32)])
    srcz = jnp.concatenate([src, jnp.full((pad,), _N, I32)])
    dstp = jnp.concatenate([dst, jnp.zeros((pad,), I32)])
    eap = jnp.concatenate([edge_attr.astype(F32), jnp.zeros((pad, 2), F32)])
    xp = jnp.concatenate([x.astype(F32), jnp.zeros((_NP - _N, _D), F32)])

    src2 = srcp.reshape(_NW * _NCHUNK, _CH)
    dst2 = dstp.reshape(_NW * _NCHUNK, _CH)
    src3 = srcp.reshape(_NW * _NCHUNK, 1, _CH)
    srcz64 = srcz.reshape(_NW * _NCHUNK2, 1, _CH2)
    src64 = srcp.reshape(_NW * _NCHUNK2, 1, _CH2)
    dst64 = dstp.reshape(_NW * _NCHUNK2, 1, _CH2)

    cd_flat, cs_flat = _sc_count(src2, dst2)
    cdT = cd_flat.reshape(_NW, _NP).T
    csT = cs_flat.reshape(_NW, _NP).T
    dinv_c, cnt_c = _tc_prep(cdT, csT)

    gcn = params["gcn"]
    egcl = params["egcl"]
    hWd = _tc_gcn_first(xp, gcn[0][0], dinv_c)
    for l in range(len(gcn)):
        part = _sc_gcn_scatter(hWd, srcz64, dst64, 0)
        part2 = _sc_gcn_scatter(hWd, srcz64, dst64, _NCH)
        bias = gcn[l][1].reshape(1, _H)
        if l < len(gcn) - 1:
            hWd = _tc_gcn_combine(part, part2, hWd, dinv_c, bias, gcn[l + 1][0])
        else:
            e0 = egcl[0]
            h, T = _tc_gcn_final(part, part2, hWd, dinv_c, bias,
                                 e0["eW1"][:_H], e0["eW1"][_H:2 * _H], xp)

    coord = xp
    for l, p in enumerate(egcl):
        pre, rad = _sc_edge_pre(T, src64, dst64)
        mp, phi = _tc_edge_mlp(
            pre, rad.reshape(_EPAD, 1), eap,
            p["eW1"][2 * _H:2 * _H + 1], p["eW1"][2 * _H + 1:],
            p["eb1"].reshape(1, _H), p["eW2"], p["eb2"].reshape(1, _H),
            p["cW1"], p["cb1"].reshape(1, _H), p["cW2"])
        mpp = _sc_scatter_m(mp, src3)
        if l < len(egcl) - 1:
            phi2 = phi.reshape(_NW * _NCHUNK2, _CH2)
            pp = _sc_scatter_p(phi2, src64, dst64, coord, 0)
            pp2 = _sc_scatter_p(phi2, src64, dst64, coord, _NCH)
            pn = egcl[l + 1]
            h, coord, T = _tc_node_full(
                h, mpp, pp, pp2, coord, cnt_c,
                p["nW1"][:_H], p["nW1"][_H:], p["nb1"].reshape(1, _H),
                p["nW2"], p["nb2"].reshape(1, _H),
                pn["eW1"][:_H], pn["eW1"][_H:2 * _H])
        else:
            h = _tc_node_last(
                h, mpp,
                p["nW1"][:_H], p["nW1"][_H:], p["nb1"].reshape(1, _H),
                p["nW2"], p["nb2"].reshape(1, _H))
    return h[:_N]


# speedup vs baseline: 2.1974x; 1.0003x over previous
import jax
import jax.numpy as jnp
from jax import lax
from jax.experimental import pallas as pl
from jax.experimental.pallas import tpu as pltpu
from jax.experimental.pallas import tpu_sc as plsc

F32 = jnp.float32
I32 = jnp.int32

_N = 10000
_E = 320000
_D = 128
_H = 64
_TW = 256

_NP = 10240
_NC = 2
_NS = 16
_NW = _NC * _NS
_CH = 128
_EP = 10240
_EPAD = _EP * _NW
_NCHUNK = _EP // _CH
_CH2 = 64
_NCHUNK2 = _EP // _CH2
_RD = _NP // _NS
_EB = 512
_NB = 1024


def _mesh():
    return plsc.VectorSubcoreMesh(core_axis_name="c", subcore_axis_name="s")


def _sc_params():
    return pltpu.CompilerParams(needs_layout_passes=False)


def _silu(v):
    return v * jax.nn.sigmoid(v)


def _worker_id():
    return lax.axis_index("c") * _NS + lax.axis_index("s")


def _sc_count(src2, dst2):
    def body(src_ref, dst_ref, outd_ref, outs_ref, sbuf, dbuf, accd, accs):
        wid = _worker_id()
        zero16 = jnp.zeros((16,), F32)
        ones16 = jnp.ones((16,), F32)
        iota16 = lax.iota(I32, 16)

        def zb(i, _):
            accd[pl.ds(i * 16, 16)] = zero16
            accs[pl.ds(i * 16, 16)] = zero16
            return 0

        lax.fori_loop(0, _NP // 16, zb, 0)
        pltpu.sync_copy(src_ref.at[pl.ds(wid * _NCHUNK, _NCHUNK)], sbuf)
        pltpu.sync_copy(dst_ref.at[pl.ds(wid * _NCHUNK, _NCHUNK)], dbuf)

        def sub(t, _):
            ci = t // (_CH // 16)
            off = (t % (_CH // 16)) * 16
            sidx = sbuf[ci, pl.ds(off, 16)]
            didx = dbuf[ci, pl.ds(off, 16)]
            gid = wid * _EP + ci * _CH + off + iota16
            val = jnp.where(gid < _E, ones16, 0.0)
            plsc.addupdate_scatter(accd, [didx], val)
            plsc.addupdate_scatter(accs, [sidx], val)
            return 0

        lax.fori_loop(0, _NCHUNK * (_CH // 16), sub, 0)
        pltpu.sync_copy(accd, outd_ref.at[pl.ds(wid * _NP, _NP)])
        pltpu.sync_copy(accs, outs_ref.at[pl.ds(wid * _NP, _NP)])

    fn = pl.kernel(
        body,
        out_type=(jax.ShapeDtypeStruct((_NW * _NP,), F32),
                  jax.ShapeDtypeStruct((_NW * _NP,), F32)),
        mesh=_mesh(),
        compiler_params=_sc_params(),
        scratch_types=[
            pltpu.VMEM((_NCHUNK, _CH), I32),
            pltpu.VMEM((_NCHUNK, _CH), I32),
            pltpu.VMEM((_NP,), F32),
            pltpu.VMEM((_NP,), F32),
        ],
    )
    return fn(src2, dst2)


def _zero_acc_rows(row0, acc, s, width, ch):
    zero16 = jnp.zeros((16,), F32)

    def zb(i, _):
        for k in range(width // 16):
            row0[i, pl.ds(k * 16, 16)] = zero16
        return 0

    lax.fori_loop(0, ch, zb, 0)

    def zcp(i, _):
        pltpu.sync_copy(row0, acc.at[pl.ds(s * _RD + i * ch, ch)])
        return 0

    lax.fori_loop(0, _RD // ch, zcp, 0)


def _gather_scatter_body(gather_src, sbuf_scatter, out_ref, rows, gsem, ssem,
                         acc, c, s, nchunk, la):
    K = len(rows)
    gd = [None] * nchunk
    sd = [None] * nchunk
    for ci in range(nchunk + la):
        if ci < nchunk:
            b = ci % K
            if ci >= K:
                sd[ci - K].wait()
            gd[ci] = pltpu.async_copy(gather_src(ci), rows[b], gsem[b])
        cj = ci - la
        if cj >= 0:
            b2 = cj % K
            gd[cj].wait()
            sd[cj] = pltpu.async_copy(
                rows[b2], acc.at[sbuf_scatter.at[cj, 0]], ssem[b2], add=True)
    for cj in range(nchunk - K, nchunk):
        sd[cj].wait()
    plsc.subcore_barrier()
    pltpu.sync_copy(acc.at[pl.ds(s * _RD, _RD)],
                    out_ref.at[c, pl.ds(s * _RD, _RD)])


_NCH = _NCHUNK2 // 2


def _sc_gcn_scatter(hWd, srcz64, dst64, off):
    def body(hw_ref, src_ref, dst_ref, out_ref,
             sbuf, dbuf, r0, r1, r2, acc, g0, g1, g2, s0, s1, s2):
        c = lax.axis_index("c")
        s = lax.axis_index("s")
        wid = c * _NS + s
        _zero_acc_rows(r0, acc, s, _D, _CH2)
        pltpu.sync_copy(src_ref.at[pl.ds(wid * _NCHUNK2 + off, _NCH)], sbuf)
        pltpu.sync_copy(dst_ref.at[pl.ds(wid * _NCHUNK2 + off, _NCH)], dbuf)
        plsc.subcore_barrier()
        _gather_scatter_body(
            lambda ci: hw_ref.at[sbuf.at[ci, 0]], dbuf, out_ref,
            [r0, r1, r2], [g0, g1, g2], [s0, s1, s2], acc, c, s, _NCH, 2)

    fn = pl.kernel(
        body,
        out_type=jax.ShapeDtypeStruct((_NC, _NP, _D), F32),
        mesh=_mesh(),
        compiler_params=_sc_params(),
        scratch_types=[
            pltpu.VMEM((_NCH, 1, _CH2), I32),
            pltpu.VMEM((_NCH, 1, _CH2), I32),
            pltpu.VMEM((_CH2, _D), F32),
            pltpu.VMEM((_CH2, _D), F32),
            pltpu.VMEM((_CH2, _D), F32),
            pltpu.VMEM_SHARED((_NP, _D), F32),
        ] + [pltpu.SemaphoreType.DMA] * 6,
    )
    return fn(hWd, srcz64, dst64)


def _sc_scatter_m(mp, src3):
    def body(mp_ref, src_ref, out_ref, sbuf, r0, r1, acc, g0, g1, s0, s1):
        c = lax.axis_index("c")
        s = lax.axis_index("s")
        wid = c * _NS + s
        _zero_acc_rows(r0, acc, s, _D, _CH)
        pltpu.sync_copy(src_ref.at[pl.ds(wid * _NCHUNK, _NCHUNK)], sbuf)
        plsc.subcore_barrier()
        _gather_scatter_body(
            lambda ci: mp_ref.at[pl.ds(wid * _EP + ci * _CH, _CH)], sbuf,
            out_ref, [r0, r1], [g0, g1], [s0, s1], acc, c, s, _NCHUNK, 1)

    fn = pl.kernel(
        body,
        out_type=jax.ShapeDtypeStruct((_NC, _NP, _D), F32),
        mesh=_mesh(),
        compiler_params=_sc_params(),
        scratch_types=[
            pltpu.VMEM((_NCHUNK, 1, _CH), I32),
            pltpu.VMEM((_CH, _D), F32),
            pltpu.VMEM((_CH, _D), F32),
            pltpu.VMEM_SHARED((_NP, _D), F32),
        ] + [pltpu.SemaphoreType.DMA] * 4,
    )
    return fn(mp, src3)


def _sc_edge_pre(TC3, src64, dst64):
    def body(t_ref, src_ref, dst_ref, pre_ref, rad_ref,
             sbuf, dbuf, ts0, ts1, td0, td1, prebuf, radbuf,
             gs0, gs1, gd0, gd1):
        wid = _worker_id()
        iota16 = lax.iota(I32, 16)
        ts = [ts0, ts1]
        td = [td0, td1]
        gs = [gs0, gs1]
        gdm = [gd0, gd1]
        pltpu.sync_copy(src_ref.at[pl.ds(wid * _NCHUNK2, _NCHUNK2)], sbuf)
        pltpu.sync_copy(dst_ref.at[pl.ds(wid * _NCHUNK2, _NCHUNK2)], dbuf)
        for b in range(2):
            pltpu.async_copy(t_ref.at[sbuf.at[b, 0]], ts[b], gs[b])
            pltpu.async_copy(t_ref.at[dbuf.at[b, 0]], td[b], gdm[b])

        def outer(g, _):
            for b in range(2):
                ci = g * 2 + b
                pltpu.make_async_copy(t_ref.at[sbuf.at[0, 0]], ts[b], gs[b]).wait()
                pltpu.make_async_copy(t_ref.at[dbuf.at[0, 0]], td[b], gdm[b]).wait()

                def sub(cc, _):
                    rvec = jnp.zeros((16,), F32)
                    for j in range(16):
                        e = cc * 16 + j
                        for k in range(_H // 16):
                            prebuf[b * _CH2 + e, pl.ds(k * 16, 16)] = (
                                ts[b][e, pl.ds(k * 16, 16)]
                                + td[b][e, pl.ds(_H + k * 16, 16)])
                        acc = jnp.zeros((16,), F32)
                        for k in range(_D // 16):
                            sl = pl.ds(2 * _H + k * 16, 16)
                            d = ts[b][e, sl] - td[b][e, sl]
                            acc = acc + d * d
                        rvec = jnp.where(iota16 == j, jnp.sum(acc), rvec)
                    radbuf[pl.ds(b * _CH2 + cc * 16, 16)] = rvec
                    return 0

                lax.fori_loop(0, _CH2 // 16, sub, 0)
                nxt = jnp.minimum(ci + 2, _NCHUNK2 - 1)
                pltpu.async_copy(t_ref.at[sbuf.at[nxt, 0]], ts[b], gs[b])
                pltpu.async_copy(t_ref.at[dbuf.at[nxt, 0]], td[b], gdm[b])
            pltpu.sync_copy(prebuf,
                            pre_ref.at[pl.ds(wid * _EP + g * 2 * _CH2,
                                             2 * _CH2)])
            pltpu.sync_copy(radbuf,
                            rad_ref.at[pl.ds(wid * _EP + g * 2 * _CH2,
                                             2 * _CH2)])
            return 0

        lax.fori_loop(0, _NCHUNK2 // 2, outer, 0)
        for b in range(2):
            pltpu.make_async_copy(t_ref.at[sbuf.at[0, 0]], ts[b], gs[b]).wait()
            pltpu.make_async_copy(t_ref.at[dbuf.at[0, 0]], td[b], gdm[b]).wait()

    fn = pl.kernel(
        body,
        out_type=(jax.ShapeDtypeStruct((_EPAD, _H), F32),
                  jax.ShapeDtypeStruct((_EPAD,), F32)),
        mesh=_mesh(),
        compiler_params=_sc_params(),
        scratch_types=[
            pltpu.VMEM((_NCHUNK2, 1, _CH2), I32),
            pltpu.VMEM((_NCHUNK2, 1, _CH2), I32),
            pltpu.VMEM((_CH2, _TW), F32),
            pltpu.VMEM((_CH2, _TW), F32),
            pltpu.VMEM((_CH2, _TW), F32),
            pltpu.VMEM((_CH2, _TW), F32),
            pltpu.VMEM((2 * _CH2, _H), F32),
            pltpu.VMEM((2 * _CH2,), F32),
        ] + [pltpu.SemaphoreType.DMA] * 4,
    )
    return fn(TC3, src64, dst64)


def _sc_scatter_p(phi2, src3, dst3, C, off):
    def body(phi_ref, src_ref, dst_ref, c_ref, out_ref,
             sbuf, dbuf, pbuf, r0, r1, acc, g0, g1, s0, s1):
        c = lax.axis_index("c")
        s = lax.axis_index("s")
        wid = c * _NS + s
        rows = [r0, r1]
        gsem = [g0, g1]
        ssem = [s0, s1]
        _zero_acc_rows(r0, acc, s, _D, _CH2)
        pltpu.sync_copy(src_ref.at[pl.ds(wid * _NCHUNK2 + off, _NCH)], sbuf)
        pltpu.sync_copy(dst_ref.at[pl.ds(wid * _NCHUNK2 + off, _NCH)], dbuf)
        pltpu.sync_copy(phi_ref.at[pl.ds(wid * _NCHUNK2 + off, _NCH)], pbuf)
        plsc.subcore_barrier()
        for b in range(2):
            pltpu.async_copy(c_ref.at[dbuf.at[b, 0]], rows[b], gsem[b])

        def outer(g, _):
            for b in range(2):
                ci = g * 2 + b
                pltpu.make_async_copy(c_ref.at[dbuf.at[0, 0]], rows[b],
                                      gsem[b]).wait()

                def scale(e, _):
                    pv = plsc.load_gather(
                        pbuf, [jnp.zeros((16,), I32) + ci,
                               jnp.zeros((16,), I32) + e])
                    for k in range(_D // 16):
                        sl = pl.ds(k * 16, 16)
                        rows[b][e, sl] = rows[b][e, sl] * pv
                    return 0

                lax.fori_loop(0, _CH2, scale, 0)
                pltpu.async_copy(rows[b], acc.at[sbuf.at[ci, 0]], ssem[b],
                                 add=True).wait()
                nxt = jnp.minimum(ci + 2, _NCH - 1)
                pltpu.async_copy(c_ref.at[dbuf.at[nxt, 0]], rows[b], gsem[b])
            return 0

        lax.fori_loop(0, _NCH // 2, outer, 0)
        for b in range(2):
            pltpu.make_async_copy(c_ref.at[dbuf.at[0, 0]], rows[b],
                                  gsem[b]).wait()
        plsc.subcore_barrier()
        pltpu.sync_copy(acc.at[pl.ds(s * _RD, _RD)],
                        out_ref.at[c, pl.ds(s * _RD, _RD)])

    fn = pl.kernel(
        body,
        out_type=jax.ShapeDtypeStruct((_NC, _NP, _D), F32),
        mesh=_mesh(),
        compiler_params=_sc_params(),
        scratch_types=[
            pltpu.VMEM((_NCH, 1, _CH2), I32),
            pltpu.VMEM((_NCH, 1, _CH2), I32),
            pltpu.VMEM((_NCH, _CH2), F32),
            pltpu.VMEM((_CH2, _D), F32),
            pltpu.VMEM((_CH2, _D), F32),
            pltpu.VMEM_SHARED((_NP, _D), F32),
        ] + [pltpu.SemaphoreType.DMA] * 4,
    )
    return fn(phi2, src3, dst3, C)


def _dot(a, b):
    return jnp.dot(a, b, preferred_element_type=F32)


def _row_valid(shape):
    rows = pl.program_id(0) * shape[0] + lax.broadcasted_iota(I32, shape, 0)
    return rows < _N


def _tc_gcn_first(x, W, dinv):
    def body(x_ref, w_ref, d_ref, o_ref):
        hw = _dot(x_ref[...], w_ref[...]) * d_ref[...]
        o_ref[...] = jnp.where(_row_valid((_NB, _D)),
                               jnp.concatenate([hw, jnp.zeros((_NB, _H), F32)],
                                               axis=1), 0.0)

    return pl.pallas_call(
        body,
        grid=(_NP // _NB,),
        in_specs=[pl.BlockSpec((_NB, _D), lambda i: (i, 0)),
                  pl.BlockSpec((_D, _H), lambda i: (0, 0)),
                  pl.BlockSpec((_NB, 1), lambda i: (i, 0))],
        out_specs=pl.BlockSpec((_NB, _D), lambda i: (i, 0)),
        out_shape=jax.ShapeDtypeStruct((_NP, _D), F32),
    )(x, W, dinv)


def _tc_prep(cdT, csT):
    def body(cd_ref, cs_ref, dinv_ref, cnt_ref):
        deg = jnp.sum(cd_ref[...], axis=1, keepdims=True) + 1.0
        dinv_ref[...] = lax.rsqrt(deg)
        cnt_ref[...] = jnp.maximum(jnp.sum(cs_ref[...], axis=1, keepdims=True), 1.0)

    return pl.pallas_call(
        body,
        grid=(_NP // _NB,),
        in_specs=[pl.BlockSpec((_NB, _NW), lambda i: (i, 0)),
                  pl.BlockSpec((_NB, _NW), lambda i: (i, 0))],
        out_specs=[pl.BlockSpec((_NB, 1), lambda i: (i, 0))] * 2,
        out_shape=[jax.ShapeDtypeStruct((_NP, 1), F32)] * 2,
    )(cdT, csT)


def _tc_gcn_combine(part, part2, hWd, dinv, b, Wnext):
    def body(p_ref, q_ref, hw_ref, d_ref, b_ref, w_ref, o_ref):
        pre = (p_ref[0, :, : _H] + p_ref[1, :, : _H]
               + q_ref[0, :, : _H] + q_ref[1, :, : _H]
               + hw_ref[:, : _H]) * d_ref[...]
        h = jnp.maximum(pre + b_ref[...], 0.0)
        hw = _dot(h, w_ref[...]) * d_ref[...]
        o_ref[...] = jnp.where(_row_valid((_NB, _D)),
                               jnp.concatenate([hw, jnp.zeros((_NB, _H), F32)],
                                               axis=1), 0.0)

    return pl.pallas_call(
        body,
        grid=(_NP // _NB,),
        in_specs=[pl.BlockSpec((_NC, _NB, _D), lambda i: (0, i, 0)),
                  pl.BlockSpec((_NC, _NB, _D), lambda i: (0, i, 0)),
                  pl.BlockSpec((_NB, _D), lambda i: (i, 0)),
                  pl.BlockSpec((_NB, 1), lambda i: (i, 0)),
                  pl.BlockSpec((1, _H), lambda i: (0, 0)),
                  pl.BlockSpec((_H, _H), lambda i: (0, 0))],
        out_specs=pl.BlockSpec((_NB, _D), lambda i: (i, 0)),
        out_shape=jax.ShapeDtypeStruct((_NP, _D), F32),
    )(part, part2, hWd, dinv, b, Wnext)


def _tc_gcn_final(part, part2, hWd, dinv, b, WA, WB, coord):
    def body(p_ref, q_ref, hw_ref, d_ref, b_ref, wa_ref, wb_ref, co_ref,
             h_ref, t_ref):
        pre = (p_ref[0, :, : _H] + p_ref[1, :, : _H]
               + q_ref[0, :, : _H] + q_ref[1, :, : _H]
               + hw_ref[:, : _H]) * d_ref[...]
        h = pre + b_ref[...]
        h_ref[...] = h
        t_ref[...] = jnp.concatenate(
            [_dot(h, wa_ref[...]), _dot(h, wb_ref[...]), co_ref[...]], axis=1)

    return pl.pallas_call(
        body,
        grid=(_NP // _NB,),
        in_specs=[pl.BlockSpec((_NC, _NB, _D), lambda i: (0, i, 0)),
                  pl.BlockSpec((_NC, _NB, _D), lambda i: (0, i, 0)),
                  pl.BlockSpec((_NB, _D), lambda i: (i, 0)),
                  pl.BlockSpec((_NB, 1), lambda i: (i, 0)),
                  pl.BlockSpec((1, _H), lambda i: (0, 0)),
                  pl.BlockSpec((_H, _H), lambda i: (0, 0)),
                  pl.BlockSpec((_H, _H), lambda i: (0, 0)),
                  pl.BlockSpec((_NB, _D), lambda i: (i, 0))],
        out_specs=[pl.BlockSpec((_NB, _H), lambda i: (i, 0)),
                   pl.BlockSpec((_NB, _TW), lambda i: (i, 0))],
        out_shape=[jax.ShapeDtypeStruct((_NP, _H), F32),
                   jax.ShapeDtypeStruct((_NP, _TW), F32)],
    )(part, part2, hWd, dinv, b, WA, WB, coord)


def _tc_edge_mlp(pre, rad, ea, wr, wea, eb1, eW2, eb2, cW1, cb1, cW2):
    def body(pre_ref, rad_ref, ea_ref, wr_ref, wea_ref, eb1_ref,
             ew2_ref, eb2_ref, cw1_ref, cb1_ref, cw2_ref, m_ref, phi_ref):
        eab = ea_ref[...]
        t = (pre_ref[...] + rad_ref[...] * wr_ref[...]
             + eab[:, 0:1] * wea_ref[0:1, :] + eab[:, 1:2] * wea_ref[1:2, :]
             + eb1_ref[...])
        m = _silu(_dot(_silu(t), ew2_ref[...]) + eb2_ref[...])
        phi = _dot(_silu(_dot(m, cw1_ref[...]) + cb1_ref[...]), cw2_ref[...])
        valid = (pl.program_id(0) * _EB
                 + lax.broadcasted_iota(I32, (_EB, 1), 0)) < _E
        phiz = jnp.where(valid, phi, 0.0)
        m_ref[...] = jnp.where(
            valid, jnp.concatenate(
                [m, phi, jnp.zeros((_EB, _D - _H - 1), F32)], axis=1), 0.0)
        phi_ref[...] = phiz

    return pl.pallas_call(
        body,
        grid=(_EPAD // _EB,),
        in_specs=[pl.BlockSpec((_EB, _H), lambda i: (i, 0)),
                  pl.BlockSpec((_EB, 1), lambda i: (i, 0)),
                  pl.BlockSpec((_EB, 2), lambda i: (i, 0)),
                  pl.BlockSpec((1, _H), lambda i: (0, 0)),
                  pl.BlockSpec((2, _H), lambda i: (0, 0)),
                  pl.BlockSpec((1, _H), lambda i: (0, 0)),
                  pl.BlockSpec((_H, _H), lambda i: (0, 0)),
                  pl.BlockSpec((1, _H), lambda i: (0, 0)),
                  pl.BlockSpec((_H, _H), lambda i: (0, 0)),
                  pl.BlockSpec((1, _H), lambda i: (0, 0)),
                  pl.BlockSpec((_H, 1), lambda i: (0, 0))],
        out_specs=[pl.BlockSpec((_EB, _D), lambda i: (i, 0)),
                   pl.BlockSpec((_EB, 1), lambda i: (i, 0))],
        out_shape=[jax.ShapeDtypeStruct((_EPAD, _D), F32),
                   jax.ShapeDtypeStruct((_EPAD, 1), F32)],
    )(pre, rad, ea, wr, wea, eb1, eW2, eb2, cW1, cb1, cW2)


def _tc_node_full(h, mpp, pp, pp2, coord, cnt, w1a, w1b, b1, w2, b2, WA, WB):
    def body(h_ref, mpp_ref, pp_ref, pq_ref, co_ref, cnt_ref,
             w1a_ref, w1b_ref, b1_ref, w2_ref, b2_ref, wa_ref, wb_ref,
             hn_ref, con_ref, t_ref):
        h = h_ref[...]
        mps = mpp_ref[0] + mpp_ref[1]
        agg = mps[:, : _H]
        S = mps[:, _H: _H + 1]
        u = _dot(_silu(_dot(h, w1a_ref[...]) + _dot(agg, w1b_ref[...])
                       + b1_ref[...]), w2_ref[...]) + b2_ref[...]
        hn = h + u
        hn_ref[...] = hn
        P = pp_ref[0] + pp_ref[1] + pq_ref[0] + pq_ref[1]
        co = co_ref[...]
        con = co + (co * S - P) / cnt_ref[...]
        con_ref[...] = con
        t_ref[...] = jnp.concatenate(
            [_dot(hn, wa_ref[...]), _dot(hn, wb_ref[...]), con], axis=1)

    return pl.pallas_call(
        body,
        grid=(_NP // _NB,),
        in_specs=[pl.BlockSpec((_NB, _H), lambda i: (i, 0)),
                  pl.BlockSpec((_NC, _NB, _D), lambda i: (0, i, 0)),
                  pl.BlockSpec((_NC, _NB, _D), lambda i: (0, i, 0)),
                  pl.BlockSpec((_NC, _NB, _D), lambda i: (0, i, 0)),
                  pl.BlockSpec((_NB, _D), lambda i: (i, 0)),
                  pl.BlockSpec((_NB, 1), lambda i: (i, 0)),
                  pl.BlockSpec((_H, _H), lambda i: (0, 0)),
                  pl.BlockSpec((_H, _H), lambda i: (0, 0)),
                  pl.BlockSpec((1, _H), lambda i: (0, 0)),
                  pl.BlockSpec((_H, _H), lambda i: (0, 0)),
                  pl.BlockSpec((1, _H), lambda i: (0, 0)),
                  pl.BlockSpec((_H, _H), lambda i: (0, 0)),
                  pl.BlockSpec((_H, _H), lambda i: (0, 0))],
        out_specs=[pl.BlockSpec((_NB, _H), lambda i: (i, 0)),
                   pl.BlockSpec((_NB, _D), lambda i: (i, 0)),
                   pl.BlockSpec((_NB, _TW), lambda i: (i, 0))],
        out_shape=[jax.ShapeDtypeStruct((_NP, _H), F32),
                   jax.ShapeDtypeStruct((_NP, _D), F32),
                   jax.ShapeDtypeStruct((_NP, _TW), F32)],
    )(h, mpp, pp, pp2, coord, cnt, w1a, w1b, b1, w2, b2, WA, WB)


def _tc_node_last(h, mpp, w1a, w1b, b1, w2, b2):
    def body(h_ref, mpp_ref, w1a_ref, w1b_ref, b1_ref, w2_ref, b2_ref, hn_ref):
        h = h_ref[...]
        agg = (mpp_ref[0] + mpp_ref[1])[:, : _H]
        u = _dot(_silu(_dot(h, w1a_ref[...]) + _dot(agg, w1b_ref[...])
                       + b1_ref[...]), w2_ref[...]) + b2_ref[...]
        hn_ref[...] = h + u

    return pl.pallas_call(
        body,
        grid=(_NP // _NB,),
        in_specs=[pl.BlockSpec((_NB, _H), lambda i: (i, 0)),
                  pl.BlockSpec((_NC, _NB, _D), lambda i: (0, i, 0)),
                  pl.BlockSpec((_H, _H), lambda i: (0, 0)),
                  pl.BlockSpec((_H, _H), lambda i: (0, 0)),
                  pl.BlockSpec((1, _H), lambda i: (0, 0)),
                  pl.BlockSpec((_H, _H), lambda i: (0, 0)),
                  pl.BlockSpec((1, _H), lambda i: (0, 0))],
        out_specs=pl.BlockSpec((_NB, _H), lambda i: (i, 0)),
        out_shape=jax.ShapeDtypeStruct((_NP, _H), F32),
    )(h, mpp, w1a, w1b, b1, w2, b2)


def kernel(x, edge_index, edge_attr, params):
    src = edge_index[0].astype(I32)
    dst = edge_index[1].astype(I32)
    pad = _EPAD - _E
    srcp = jnp.concatenate([src, jnp.zeros((pad,), I32)])
    srcz = jnp.concatenate([src, jnp.full((pad,), _N, I32)])
    dstp = jnp.concatenate([dst, jnp.zeros((pad,), I32)])
    eap = jnp.concatenate([edge_attr.astype(F32), jnp.zeros((pad, 2), F32)])
    xp = jnp.concatenate([x.astype(F32), jnp.zeros((_NP - _N, _D), F32)])

    src2 = srcp.reshape(_NW * _NCHUNK, _CH)
    dst2 = dstp.reshape(_NW * _NCHUNK, _CH)
    src3 = srcp.reshape(_NW * _NCHUNK, 1, _CH)
    srcz64 = srcz.reshape(_NW * _NCHUNK2, 1, _CH2)
    src64 = srcp.reshape(_NW * _NCHUNK2, 1, _CH2)
    dst64 = dstp.reshape(_NW * _NCHUNK2, 1, _CH2)

    cd_flat, cs_flat = _sc_count(src2, dst2)
    cdT = cd_flat.reshape(_NW, _NP).T
    csT = cs_flat.reshape(_NW, _NP).T
    dinv_c, cnt_c = _tc_prep(cdT, csT)

    gcn = params["gcn"]
    egcl = params["egcl"]
    hWd = _tc_gcn_first(xp, gcn[0][0], dinv_c)
    for l in range(len(gcn)):
        part = _sc_gcn_scatter(hWd, srcz64, dst64, 0)
        part2 = _sc_gcn_scatter(hWd, srcz64, dst64, _NCH)
        bias = gcn[l][1].reshape(1, _H)
        if l < len(gcn) - 1:
            hWd = _tc_gcn_combine(part, part2, hWd, dinv_c, bias, gcn[l + 1][0])
        else:
            e0 = egcl[0]
            h, T = _tc_gcn_final(part, part2, hWd, dinv_c, bias,
                                 e0["eW1"][:_H], e0["eW1"][_H:2 * _H], xp)

    coord = xp
    for l, p in enumerate(egcl):
        pre, rad = _sc_edge_pre(T, src64, dst64)
        mp, phi = _tc_edge_mlp(
            pre, rad.reshape(_EPAD, 1), eap,
            p["eW1"][2 * _H:2 * _H + 1], p["eW1"][2 * _H + 1:],
            p["eb1"].reshape(1, _H), p["eW2"], p["eb2"].reshape(1, _H),
            p["cW1"], p["cb1"].reshape(1, _H), p["cW2"])
        mpp = _sc_scatter_m(mp, src3)
        if l < len(egcl) - 1:
            phi2 = phi.reshape(_NW * _NCHUNK2, _CH2)
            pp = _sc_scatter_p(phi2, src64, dst64, coord, 0)
            pp2 = _sc_scatter_p(phi2, src64, dst64, coord, _NCH)
            pn = egcl[l + 1]
            h, coord, T = _tc_node_full(
                h, mpp, pp, pp2, coord, cnt_c,
                p["nW1"][:_H], p["nW1"][_H:], p["nb1"].reshape(1, _H),
                p["nW2"], p["nb2"].reshape(1, _H),
                pn["eW1"][:_H], pn["eW1"][_H:2 * _H])
        else:
            h = _tc_node_last(
                h, mpp,
                p["nW1"][:_H], p["nW1"][_H:], p["nb1"].reshape(1, _H),
                p["nW2"], p["nb2"].reshape(1, _H))
    return h[:_N]
